# Initial kernel scaffold; baseline (speedup 1.0000x reference)
import jax
import jax.numpy as jnp
from jax.experimental import pallas as pl


def kernel(x, adj, idx, W1, b1, W2, b2, d1_W, d1_b, d2_W, d2_b):
    raise NotImplementedError("write your pallas kernel here")


# speedup vs baseline: 4.3072x; 4.3072x over previous
import functools

import jax
import jax.numpy as jnp
from jax import lax
from jax.experimental import pallas as pl
from jax.experimental.pallas import tpu as pltpu
from jax.experimental.pallas import tpu_sc as plsc

_N = 10000
_P = 131072
_BM = 400
_NC, _NS, _L = 2, 16, 16
_NW = _NC * _NS
_BP = _P // _NW


def _layer1_body(x_ref, w1_ref, adj_ref, b1_ref, wv_ref, t_ref, g_ref):
    @pl.when(pl.program_id(0) == 0)
    def _():
        g_ref[...] = jnp.dot(x_ref[...], w1_ref[...],
                             preferred_element_type=jnp.float32)

    h = jnp.dot(adj_ref[...], g_ref[...], preferred_element_type=jnp.float32)
    h = jnp.maximum(h + b1_ref[...], 0.0)
    t_ref[...] = jnp.dot(h, wv_ref[...], preferred_element_type=jnp.float32)


def _layer2_body(adj_ref, t_ref, c2_ref, s_ref):
    s = jnp.dot(adj_ref[...], t_ref[...], preferred_element_type=jnp.float32)
    s_ref[...] = s + c2_ref[...]


def _sc_gather_body(s_hbm, i0_hbm, i1_hbm, out_hbm, s_v, i0_v, i1_v, o_v):
    wid = lax.axis_index("s") * _NC + lax.axis_index("c")
    base = wid * _BP
    pltpu.sync_copy(s_hbm, s_v)
    pltpu.sync_copy(i0_hbm.at[pl.ds(base, _BP)], i0_v)
    pltpu.sync_copy(i1_hbm.at[pl.ds(base, _BP)], i1_v)

    ones = jnp.ones((_L,), jnp.int32)

    def body(i, carry):
        i0 = i0_v[pl.ds(i * _L, _L)]
        i1 = i1_v[pl.ds(i * _L, _L)]
        a = plsc.load_gather(s_v, [i0 * 2])
        b = plsc.load_gather(s_v, [i1 * 2 + ones])
        o_v[pl.ds(i * _L, _L)] = a + b
        return carry

    lax.fori_loop(0, _BP // _L, body, 0)
    pltpu.sync_copy(o_v, out_hbm.at[pl.ds(base, _BP)])


def kernel(x, adj, idx, W1, b1, W2, b2, d1_W, d1_b, d2_W, d2_b):
    nhid2 = W2.shape[1]
    v12 = jnp.concatenate([d1_W[:nhid2] @ d2_W, d1_W[nhid2:] @ d2_W], axis=1)
    wv = W2 @ v12
    c = d1_b @ d2_W + d2_b
    c2 = b2 @ v12 + jnp.concatenate([c, jnp.zeros((1,), jnp.float32)])

    grid = _N // _BM
    t = pl.pallas_call(
        _layer1_body,
        grid=(grid,),
        in_specs=[
            pl.BlockSpec((_N, 128), lambda i: (0, 0)),
            pl.BlockSpec((128, 128), lambda i: (0, 0)),
            pl.BlockSpec((_BM, _N), lambda i: (i, 0)),
            pl.BlockSpec((1, 128), lambda i: (0, 0)),
            pl.BlockSpec((128, 2), lambda i: (0, 0)),
        ],
        out_specs=pl.BlockSpec((_BM, 2), lambda i: (i, 0)),
        out_shape=jax.ShapeDtypeStruct((_N, 2), jnp.float32),
        scratch_shapes=[pltpu.VMEM((_N, 128), jnp.float32)],
    )(x, W1, adj, b1.reshape(1, 128), wv)

    s = pl.pallas_call(
        _layer2_body,
        grid=(grid,),
        in_specs=[
            pl.BlockSpec((_BM, _N), lambda i: (i, 0)),
            pl.BlockSpec((_N, 2), lambda i: (0, 0)),
            pl.BlockSpec((1, 2), lambda i: (0, 0)),
        ],
        out_specs=pl.BlockSpec((_BM, 2), lambda i: (i, 0)),
        out_shape=jax.ShapeDtypeStruct((_N, 2), jnp.float32),
    )(adj, t, c2.reshape(1, 2))

    mesh = plsc.VectorSubcoreMesh(core_axis_name="c", subcore_axis_name="s",
                                  num_cores=_NC, num_subcores=_NS)
    o = pl.kernel(
        _sc_gather_body,
        out_type=jax.ShapeDtypeStruct((_P,), jnp.float32),
        mesh=mesh,
        compiler_params=pltpu.CompilerParams(needs_layout_passes=False),
        scratch_types=[
            pltpu.VMEM((2 * _N,), jnp.float32),
            pltpu.VMEM((_BP,), jnp.int32),
            pltpu.VMEM((_BP,), jnp.int32),
            pltpu.VMEM((_BP,), jnp.float32),
        ],
    )(s.reshape(2 * _N), idx[0], idx[1])

    return o.reshape(_P, 1)


# speedup vs baseline: 4.3265x; 1.0045x over previous
import functools

import jax
import jax.numpy as jnp
from jax import lax
from jax.experimental import pallas as pl
from jax.experimental.pallas import tpu as pltpu
from jax.experimental.pallas import tpu_sc as plsc

_N = 10000
_P = 131072
_BM = 400
_NC, _NS, _L = 2, 16, 16
_NW = _NC * _NS
_BP = _P // _NW


def _gcn_body(x_ref, w1_ref, adj_ref, b1_ref, wv_ref, c2_ref, s_ref,
              g_ref, t_ref):
    p = pl.program_id(0)
    i = pl.program_id(1)

    @pl.when((p == 0) & (i == 0))
    def _():
        g_ref[...] = jnp.dot(x_ref[...], w1_ref[...],
                             preferred_element_type=jnp.float32)

    @pl.when(p == 0)
    def _():
        h = jnp.dot(adj_ref[...], g_ref[...],
                    preferred_element_type=jnp.float32)
        h = jnp.maximum(h + b1_ref[...], 0.0)
        t_ref[pl.ds(i * _BM, _BM), :] = jnp.dot(
            h, wv_ref[...], preferred_element_type=jnp.float32)

    @pl.when(p == 1)
    def _():
        s = jnp.dot(adj_ref[...], t_ref[...],
                    preferred_element_type=jnp.float32)
        s_ref[...] = s + c2_ref[...]


def _sc_gather_body(s_hbm, i0_hbm, i1_hbm, out_hbm, s_v, i0_v, i1_v, o_v):
    wid = lax.axis_index("s") * _NC + lax.axis_index("c")
    base = wid * _BP
    pltpu.sync_copy(s_hbm, s_v)
    pltpu.sync_copy(i0_hbm.at[pl.ds(base, _BP)], i0_v)
    pltpu.sync_copy(i1_hbm.at[pl.ds(base, _BP)], i1_v)

    ones = jnp.ones((_L,), jnp.int32)

    def body(i, carry):
        i0 = i0_v[pl.ds(i * _L, _L)]
        i1 = i1_v[pl.ds(i * _L, _L)]
        a = plsc.load_gather(s_v, [i0 * 2])
        b = plsc.load_gather(s_v, [i1 * 2 + ones])
        o_v[pl.ds(i * _L, _L)] = a + b
        return carry

    lax.fori_loop(0, _BP // _L, body, 0)
    pltpu.sync_copy(o_v, out_hbm.at[pl.ds(base, _BP)])


def kernel(x, adj, idx, W1, b1, W2, b2, d1_W, d1_b, d2_W, d2_b):
    nhid2 = W2.shape[1]
    v12 = jnp.concatenate([d1_W[:nhid2] @ d2_W, d1_W[nhid2:] @ d2_W], axis=1)
    wv = W2 @ v12
    c = d1_b @ d2_W + d2_b
    c2 = b2 @ v12 + jnp.concatenate([c, jnp.zeros((1,), jnp.float32)])

    grid = _N // _BM
    s = pl.pallas_call(
        _gcn_body,
        grid=(2, grid),
        in_specs=[
            pl.BlockSpec((_N, 128), lambda p, i: (0, 0)),
            pl.BlockSpec((128, 128), lambda p, i: (0, 0)),
            pl.BlockSpec((_BM, _N), lambda p, i: (i, 0)),
            pl.BlockSpec((1, 128), lambda p, i: (0, 0)),
            pl.BlockSpec((128, 2), lambda p, i: (0, 0)),
            pl.BlockSpec((1, 2), lambda p, i: (0, 0)),
        ],
        out_specs=pl.BlockSpec((_BM, 2), lambda p, i: (i, 0)),
        out_shape=jax.ShapeDtypeStruct((_N, 2), jnp.float32),
        scratch_shapes=[pltpu.VMEM((_N, 128), jnp.float32),
                        pltpu.VMEM((_N, 2), jnp.float32)],
    )(x, W1, adj, b1.reshape(1, 128), wv, c2.reshape(1, 2))

    mesh = plsc.VectorSubcoreMesh(core_axis_name="c", subcore_axis_name="s",
                                  num_cores=_NC, num_subcores=_NS)
    o = pl.kernel(
        _sc_gather_body,
        out_type=jax.ShapeDtypeStruct((_P,), jnp.float32),
        mesh=mesh,
        compiler_params=pltpu.CompilerParams(needs_layout_passes=False),
        scratch_types=[
            pltpu.VMEM((2 * _N,), jnp.float32),
            pltpu.VMEM((_BP,), jnp.int32),
            pltpu.VMEM((_BP,), jnp.int32),
            pltpu.VMEM((_BP,), jnp.float32),
        ],
    )(s.reshape(2 * _N), idx[0], idx[1])

    return o.reshape(_P, 1)
